# baseline (speedup 1.0000x reference)
import functools

import jax
import jax.numpy as jnp
from jax import lax
from jax.experimental import pallas as pl
from jax.experimental.pallas import tpu as pltpu
from jax.experimental.pallas import tpu_sc as plsc

DIM = 64
CHUNK = 128


@functools.partial(jax.jit, static_argnums=(3, 4))
def _embed_sc(weight_pad, scale, ids_t3, n_chunks, n_workers):
    mesh = plsc.VectorSubcoreMesh(core_axis_name="c", subcore_axis_name="s")
    assert n_chunks % 2 == 0

    @functools.partial(
        pl.kernel,
        mesh=mesh,
        compiler_params=pltpu.CompilerParams(
            needs_layout_passes=False, use_tc_tiling_on_sc=True
        ),
        out_type=jax.ShapeDtypeStruct(
            (n_chunks, DIM // 8, n_workers, 8, CHUNK), jnp.float32
        ),
        scratch_types=[
            pltpu.VMEM((n_chunks, CHUNK), jnp.int32),
            pltpu.VMEM((n_chunks, CHUNK), jnp.int32),
            pltpu.VMEM((n_chunks, CHUNK), jnp.int32),
            pltpu.VMEM((CHUNK, 2 * DIM), jnp.int32),
            pltpu.VMEM((CHUNK, 2 * DIM), jnp.int32),
            pltpu.VMEM((CHUNK,), jnp.float32),
            pltpu.VMEM((CHUNK,), jnp.float32),
            pltpu.VMEM((DIM, CHUNK + 5), jnp.float32),
            pltpu.VMEM((DIM, CHUNK), jnp.float32),
            pltpu.VMEM((DIM, CHUNK), jnp.float32),
            pltpu.SemaphoreType.DMA,
            pltpu.SemaphoreType.DMA,
            pltpu.SemaphoreType.DMA,
            pltpu.SemaphoreType.DMA,
            pltpu.SemaphoreType.DMA,
            pltpu.SemaphoreType.DMA,
        ],
    )
    def k(w_hbm, s_hbm, ids_hbm, out_hbm, idx_v, idg_v, byp_v, rows0_v, rows1_v,
          sc0_v, sc1_v, skw_v, st0_v, st1_v, sem_r0, sem_r1, sem_s0, sem_s1,
          sem_w0, sem_w1):
        wid = lax.axis_index("s") * 2 + lax.axis_index("c")
        pltpu.sync_copy(ids_hbm.at[wid], idx_v)

        w32_hbm = w_hbm.bitcast(jnp.int32)
        lanes = lax.iota(jnp.int32, 16)

        def pre_body(i, carry):
            c = i // (CHUNK // 16)
            off = (i % (CHUNK // 16)) * 16
            v = idx_v[c, pl.ds(off, 16)]
            idg_v[c, pl.ds(off, 16)] = v >> 2
            byp_v[c, pl.ds(off, 16)] = (v & 3) * 8
            return carry

        lax.fori_loop(0, n_chunks * (CHUNK // 16), pre_body, 0, unroll=4)

        def gather(c, rows_v, sc_v, sem_r, sem_s):
            pltpu.async_copy(w32_hbm.at[idg_v.at[c]], rows_v, sem_r)
            pltpu.async_copy(s_hbm.at[idx_v.at[c]], sc_v, sem_s)

        def wait_gather(c, rows_v, sc_v, sem_r, sem_s):
            pltpu.make_async_copy(w32_hbm.at[idg_v.at[c]], rows_v, sem_r).wait()
            pltpu.make_async_copy(s_hbm.at[idx_v.at[c]], sc_v, sem_s).wait()

        def dequant(c, rows_v, sc_v, st_v):
            def row_body(r, carry2):
                rfull = jnp.full((16,), r, dtype=jnp.int32)
                s_bc = plsc.load_gather(sc_v, [rfull])
                cfull = jnp.full((16,), c, dtype=jnp.int32)
                sh_bc = plsc.load_gather(byp_v, [cfull, rfull])
                for j in range(4):
                    w = rows_v[r, pl.ds(16 * j, 16)]
                    b = ((w >> sh_bc) << 24) >> 24
                    plsc.store_scatter(
                        skw_v, [lanes + (16 * j), rfull],
                        b.astype(jnp.float32) * s_bc,
                    )
                return carry2

            lax.fori_loop(0, CHUNK, row_body, 0, unroll=4)

            def cpt_body(d, carry2):
                for kk in range(CHUNK // 16):
                    st_v[d, pl.ds(kk * 16, 16)] = skw_v[d, pl.ds(kk * 16, 16)]
                return carry2

            lax.fori_loop(0, DIM, cpt_body, 0, unroll=4)

        def store(c, st_v, sem_w):
            for dt in range(DIM // 8):
                pltpu.async_copy(
                    st_v.at[pl.ds(dt * 8, 8)], out_hbm.at[c, dt, wid], sem_w
                )

        def wait_store(st_v, sem_w):
            for dt in range(DIM // 8):
                pltpu.make_async_copy(
                    st_v.at[pl.ds(dt * 8, 8)], out_hbm.at[0, dt, 0], sem_w
                ).wait()

        gather(0, rows0_v, sc0_v, sem_r0, sem_s0)
        gather(1, rows1_v, sc1_v, sem_r1, sem_s1)
        wait_gather(0, rows0_v, sc0_v, sem_r0, sem_s0)
        dequant(0, rows0_v, sc0_v, st0_v)
        store(0, st0_v, sem_w0)
        gather(2, rows0_v, sc0_v, sem_r0, sem_s0)
        wait_gather(1, rows1_v, sc1_v, sem_r1, sem_s1)
        dequant(1, rows1_v, sc1_v, st1_v)
        store(1, st1_v, sem_w1)
        gather(3, rows1_v, sc1_v, sem_r1, sem_s1)

        def pair_body(c2, carry):
            c0 = c2 * 2
            c1 = c0 + 1
            wait_gather(c0, rows0_v, sc0_v, sem_r0, sem_s0)
            wait_store(st0_v, sem_w0)
            dequant(c0, rows0_v, sc0_v, st0_v)
            store(c0, st0_v, sem_w0)
            gather(lax.rem(c0 + 2, n_chunks), rows0_v, sc0_v, sem_r0, sem_s0)
            wait_gather(c1, rows1_v, sc1_v, sem_r1, sem_s1)
            wait_store(st1_v, sem_w1)
            dequant(c1, rows1_v, sc1_v, st1_v)
            store(c1, st1_v, sem_w1)
            gather(lax.rem(c1 + 2, n_chunks), rows1_v, sc1_v, sem_r1, sem_s1)
            return carry

        lax.fori_loop(1, n_chunks // 2, pair_body, 0)
        wait_gather(0, rows0_v, sc0_v, sem_r0, sem_s0)
        wait_gather(1, rows1_v, sc1_v, sem_r1, sem_s1)
        wait_store(st0_v, sem_w0)
        wait_store(st1_v, sem_w1)

    return k(weight_pad, scale, ids_t3)


def kernel(weight_int8, scale, input_ids):
    B, T = input_ids.shape
    n_workers = 32
    assert B == n_workers * CHUNK
    ids_t3 = input_ids.T.reshape(T, n_workers, CHUNK).transpose(1, 0, 2)
    w_pad = jnp.pad(weight_int8, ((0, 0), (0, 2 * DIM - weight_int8.shape[1])))
    out5 = _embed_sc(w_pad, scale, ids_t3, T, n_workers)
    return out5.transpose(2, 4, 0, 1, 3).reshape(B, T, DIM)


# speedup vs baseline: 1.2164x; 1.2164x over previous
import functools

import jax
import jax.numpy as jnp
from jax import lax
from jax.experimental import pallas as pl
from jax.experimental.pallas import tpu as pltpu
from jax.experimental.pallas import tpu_sc as plsc

DIM = 64
CHUNK = 128


@functools.partial(jax.jit, static_argnums=(3, 4))
def _embed_sc(weight_pad, scale, flat_ids3, n_chunks, n_workers):
    mesh = plsc.VectorSubcoreMesh(core_axis_name="c", subcore_axis_name="s")
    n_per_w = n_chunks * CHUNK
    total = n_per_w * n_workers
    assert n_chunks % 2 == 0

    @functools.partial(
        pl.kernel,
        mesh=mesh,
        compiler_params=pltpu.CompilerParams(
            needs_layout_passes=False, use_tc_tiling_on_sc=True
        ),
        out_type=jax.ShapeDtypeStruct((total // 2, 2 * DIM), jnp.float32),
        scratch_types=[
            pltpu.VMEM((n_chunks, CHUNK), jnp.int32),
            pltpu.VMEM((n_chunks, CHUNK), jnp.int32),
            pltpu.VMEM((n_chunks, CHUNK), jnp.int32),
            pltpu.VMEM((CHUNK, 2 * DIM), jnp.int32),
            pltpu.VMEM((CHUNK, 2 * DIM), jnp.int32),
            pltpu.VMEM((CHUNK,), jnp.float32),
            pltpu.VMEM((CHUNK,), jnp.float32),
            pltpu.VMEM((CHUNK // 2, 2 * DIM), jnp.float32),
            pltpu.VMEM((CHUNK // 2, 2 * DIM), jnp.float32),
            pltpu.SemaphoreType.DMA,
            pltpu.SemaphoreType.DMA,
            pltpu.SemaphoreType.DMA,
            pltpu.SemaphoreType.DMA,
            pltpu.SemaphoreType.DMA,
            pltpu.SemaphoreType.DMA,
        ],
    )
    def k(w_hbm, s_hbm, ids_hbm, out_hbm, idx_v, idg_v, byp_v, rows0_v, rows1_v,
          sc0_v, sc1_v, outb0_v, outb1_v, sem_r0, sem_r1, sem_s0, sem_s1,
          sem_w0, sem_w1):
        wid = lax.axis_index("s") * 2 + lax.axis_index("c")
        base = wid * n_per_w
        pltpu.sync_copy(ids_hbm.at[wid], idx_v)

        w32_hbm = w_hbm.bitcast(jnp.int32)

        def pre_body(i, carry):
            c = i // (CHUNK // 16)
            off = (i % (CHUNK // 16)) * 16
            v = idx_v[c, pl.ds(off, 16)]
            idg_v[c, pl.ds(off, 16)] = v >> 2
            byp_v[c, pl.ds(off, 16)] = (v & 3) * 8
            return carry

        lax.fori_loop(0, n_chunks * (CHUNK // 16), pre_body, 0, unroll=4)

        def gather(c, rows_v, sc_v, sem_r, sem_s):
            cp_r = pltpu.async_copy(w32_hbm.at[idg_v.at[c]], rows_v, sem_r)
            cp_s = pltpu.async_copy(s_hbm.at[idx_v.at[c]], sc_v, sem_s)
            return cp_r, cp_s

        def dequant(c, rows_v, sc_v, outb_v):
            def row_body(r, carry2):
                rfull = jnp.full((16,), r, dtype=jnp.int32)
                s_bc = plsc.load_gather(sc_v, [rfull])
                cfull = jnp.full((16,), c, dtype=jnp.int32)
                sh_bc = plsc.load_gather(byp_v, [cfull, rfull])
                srow = r >> 1
                colb = (r & 1) * DIM
                for j in range(4):
                    w = rows_v[r, pl.ds(16 * j, 16)]
                    b = ((w >> sh_bc) << 24) >> 24
                    outb_v[srow, pl.ds(colb + 16 * j, 16)] = b.astype(jnp.float32) * s_bc
                return carry2

            lax.fori_loop(0, CHUNK, row_body, 0, unroll=8)

        def store(c, outb_v, sem_w):
            srow0 = pl.multiple_of((base + c * CHUNK) // 2, CHUNK // 2)
            return pltpu.async_copy(outb_v, out_hbm.at[pl.ds(srow0, CHUNK // 2)], sem_w)

        def wait_gather(c, rows_v, sc_v, sem_r, sem_s):
            pltpu.make_async_copy(w32_hbm.at[idg_v.at[c]], rows_v, sem_r).wait()
            pltpu.make_async_copy(s_hbm.at[idx_v.at[c]], sc_v, sem_s).wait()

        def wait_store(outb_v, sem_w):
            pltpu.make_async_copy(outb_v, out_hbm.at[pl.ds(0, CHUNK // 2)], sem_w).wait()

        gather(0, rows0_v, sc0_v, sem_r0, sem_s0)
        gather(1, rows1_v, sc1_v, sem_r1, sem_s1)
        wait_gather(0, rows0_v, sc0_v, sem_r0, sem_s0)
        dequant(0, rows0_v, sc0_v, outb0_v)
        store(0, outb0_v, sem_w0)
        gather(2, rows0_v, sc0_v, sem_r0, sem_s0)
        wait_gather(1, rows1_v, sc1_v, sem_r1, sem_s1)
        dequant(1, rows1_v, sc1_v, outb1_v)
        store(1, outb1_v, sem_w1)
        gather(3, rows1_v, sc1_v, sem_r1, sem_s1)

        def pair_body(c2, carry):
            c0 = c2 * 2
            c1 = c0 + 1
            wait_gather(c0, rows0_v, sc0_v, sem_r0, sem_s0)
            wait_store(outb0_v, sem_w0)
            dequant(c0, rows0_v, sc0_v, outb0_v)
            store(c0, outb0_v, sem_w0)
            gather(lax.rem(c0 + 2, n_chunks), rows0_v, sc0_v, sem_r0, sem_s0)
            wait_gather(c1, rows1_v, sc1_v, sem_r1, sem_s1)
            wait_store(outb1_v, sem_w1)
            dequant(c1, rows1_v, sc1_v, outb1_v)
            store(c1, outb1_v, sem_w1)
            gather(lax.rem(c1 + 2, n_chunks), rows1_v, sc1_v, sem_r1, sem_s1)
            return carry

        lax.fori_loop(1, n_chunks // 2, pair_body, 0)
        wait_gather(0, rows0_v, sc0_v, sem_r0, sem_s0)
        wait_gather(1, rows1_v, sc1_v, sem_r1, sem_s1)
        wait_store(outb0_v, sem_w0)
        wait_store(outb1_v, sem_w1)

    return k(weight_pad, scale, flat_ids3)


def kernel(weight_int8, scale, input_ids):
    B, T = input_ids.shape
    n = B * T
    n_workers = 32
    assert n % (n_workers * CHUNK) == 0
    n_chunks = n // (n_workers * CHUNK)
    flat3 = input_ids.reshape(n_workers, n_chunks, CHUNK)
    w_pad = jnp.pad(weight_int8, ((0, 0), (0, 2 * DIM - weight_int8.shape[1])))
    out = _embed_sc(w_pad, scale, flat3, n_chunks, n_workers)
    return out.reshape(B, T, DIM)
